# baseline (speedup 1.0000x reference)
import functools

import jax
import jax.numpy as jnp
from jax import lax
from jax.experimental import pallas as pl
from jax.experimental.pallas import tpu as pltpu
from jax.experimental.pallas import tpu_sc as plsc

H = W = 512
N = H * W
BL = 128
NB = N // BL
NW = 32


def _scan_body(x_ref, plin_ref, bstart_ref, plast_ref, mask_ref, pix_ref,
               ptsr_ref, sdfc_ref, xt_ref, p_ref, l2_ref):
    for t in range(16):
        xt_ref[:, t, :] = jnp.swapaxes(x_ref[pl.ds(t * BL, BL), :], 0, 1)
    x = xt_ref[...]
    l_idx = lax.broadcasted_iota(jnp.int32, (BL, 16, BL), 2)
    maskb = (l_idx % 4) < 2
    valid = jnp.where((jnp.abs(x) < jnp.float32(0.1)) & maskb,
                      jnp.float32(1.0), jnp.float32(0.0))
    s_total = jnp.sum(valid)
    probs = valid / jnp.maximum(s_total, jnp.float32(1e-12))

    s = jnp.zeros((16, BL), jnp.float32)
    for j in range(BL):
        s = s + probs[j]
        p_ref[j] = s

    s2d = p_ref[127]
    s2 = jnp.zeros((16, 1), jnp.float32)
    for l in range(BL):
        s2 = s2 + s2d[:, l:l + 1]
        l2_ref[:, l:l + 1] = s2

    rs3 = l2_ref[:, 127:128]
    row_i = lax.broadcasted_iota(jnp.int32, (16, 1), 0)
    carry = jnp.float32(0.0)
    off3 = jnp.zeros((16, 1), jnp.float32)
    for k in range(16):
        carry = carry + jnp.sum(jnp.where(row_i == k, rs3, jnp.float32(0.0)))
        off3 = jnp.where(row_i == k, carry, off3)

    off2 = jnp.concatenate([jnp.zeros((1, 1), jnp.float32), off3[:15, :]], axis=0)
    off_incl = l2_ref[...] + off2
    prev_last = off_incl[:, 127:128]
    prev_shift = jnp.concatenate(
        [jnp.zeros((1, 1), jnp.float32), prev_last[:15, :]], axis=0)
    off1 = jnp.concatenate([prev_shift, off_incl[:, :127]], axis=1)
    pfin = p_ref[...] + off1[None, :, :]

    for t in range(16):
        plin_ref[pl.ds(t * BL, BL), :] = jnp.swapaxes(pfin[:, t, :], 0, 1)
    bstart_ref[...] = pfin[0]
    plast_ref[...] = jnp.broadcast_to(pfin[127:128, 15, 127:128], (1, 16))

    sdfc_ref[...] = x_ref[...]
    one = jnp.float32(1.0)
    half = jnp.float32(0.5)
    inv256 = jnp.float32(1.0 / 256.0)

    colm = lax.broadcasted_iota(jnp.int32, (H, W), 1)
    mask_ref[...] = jnp.where(colm < 256, one, jnp.float32(0.0))

    rp = lax.broadcasted_iota(jnp.int32, (H, 2 * W), 0)
    cp = lax.broadcasted_iota(jnp.int32, (H, 2 * W), 1)
    gxp = (lax.convert_element_type(lax.shift_right_logical(cp, 1),
                                    jnp.float32) + half) * inv256 - one
    gyp = (lax.convert_element_type(rp, jnp.float32) + half) * inv256 - one
    pix_ref[...] = jnp.where((cp & 1) == 1, gyp, gxp)

    rr = lax.broadcasted_iota(jnp.int32, (32, 64), 0)
    cc = lax.broadcasted_iota(jnp.int32, (32, 64), 1)
    gxr = (lax.convert_element_type(lax.shift_right_logical(cc, 1) * 16,
                                    jnp.float32) + half) * inv256 - one
    gyr = (lax.convert_element_type(rr * 16, jnp.float32) + half) * inv256 - one
    ptsr_ref[...] = jnp.where((cc & 1) == 1, gyr, gxr)


def _blocked_cumsum(sdf2d):
    return pl.pallas_call(
        _scan_body,
        out_shape=[
            jax.ShapeDtypeStruct((NB, BL), jnp.float32),
            jax.ShapeDtypeStruct((16, BL), jnp.float32),
            jax.ShapeDtypeStruct((1, 16), jnp.float32),
            jax.ShapeDtypeStruct((H, W), jnp.float32),
            jax.ShapeDtypeStruct((H, 2 * W), jnp.float32),
            jax.ShapeDtypeStruct((32, 64), jnp.float32),
            jax.ShapeDtypeStruct((NB, BL), jnp.float32),
        ],
        scratch_shapes=[pltpu.VMEM((BL, 16, BL), jnp.float32),
                        pltpu.VMEM((BL, 16, BL), jnp.float32),
                        pltpu.VMEM((16, BL), jnp.float32)],
    )(sdf2d)


@functools.lru_cache(maxsize=1)
def _sc_sample_kernel():
  mesh = plsc.VectorSubcoreMesh(core_axis_name="c", subcore_axis_name="s")
  return functools.partial(
    pl.kernel,
    mesh=mesh,
    compiler_params=pltpu.CompilerParams(
        needs_layout_passes=False, use_tc_tiling_on_sc=False),
    out_type=[
        jax.ShapeDtypeStruct((2048, 2), jnp.float32),
        jax.ShapeDtypeStruct((2048, 1), jnp.float32),
        jax.ShapeDtypeStruct((2048, 2), jnp.float32),
        jax.ShapeDtypeStruct((2048, 1), jnp.float32),
        jax.ShapeDtypeStruct((1024, 1), jnp.float32),
    ],
    scratch_types=[
        pltpu.VMEM((16, 128), jnp.float32),
        pltpu.VMEM((16,), jnp.float32),
        pltpu.VMEM((64,), jnp.float32),
        pltpu.VMEM((64,), jnp.float32),
        pltpu.VMEM((64,), jnp.int32),
        pltpu.VMEM((64, BL), jnp.float32),
        pltpu.VMEM((128,), jnp.int32),
        pltpu.VMEM((128, 16), jnp.float32),
        pltpu.VMEM((32,), jnp.int32),
        pltpu.VMEM((32, 16), jnp.float32),
        pltpu.VMEM((64, 2), jnp.float32),
        pltpu.VMEM((64, 1), jnp.float32),
        pltpu.VMEM((64, 2), jnp.float32),
        pltpu.VMEM((64, 1), jnp.float32),
        pltpu.VMEM((32, 1), jnp.float32),
        pltpu.SemaphoreType.DMA,
    ],
  )(_sc_sample_body)


def _sc_sample_body(p_lin, bstart, plast, uu, ug, sdf16,
               ptsu, sdfu, ptsg, sdfg, sdfr,
               bstart_v, plast_v, uu_v, ug_v, rowidx_v, rows_v,
               sdfidx_v, sdfrows_v, sdfidx_r, sdfrows_r,
               ptsu_b, sdfu_b, ptsg_b, sdfg_b, sdfr_b, sem):
    wid = lax.axis_index("s") * 2 + lax.axis_index("c")
    base64 = wid * 64
    pltpu.sync_copy(bstart, bstart_v)
    pltpu.sync_copy(plast, plast_v)
    pltpu.sync_copy(uu.at[pl.ds(base64, 64)], uu_v)
    pltpu.sync_copy(ug.at[pl.ds(base64, 64)], ug_v)

    lane = lax.iota(jnp.int32, 16)
    zero16 = jnp.zeros((16,), jnp.int32)
    one = jnp.float32(1.0)
    plastv = plast_v[...]
    half = jnp.float32(0.5)
    inv256 = jnp.float32(1.0 / 256.0)

    def coords(idx):
        colf = lax.convert_element_type(idx & 511, jnp.float32)
        rowf = lax.convert_element_type(lax.shift_right_logical(idx, 9),
                                        jnp.float32)
        gx = (colf + half) * inv256 - one
        gy = (rowf + half) * inv256 - one
        return gx, gy

    states = []
    for g in range(4):
        u = uu_v[pl.ds(g * 16, 16)]
        r = plastv * (one - u)
        low = jnp.zeros((16,), jnp.int32)
        high = jnp.full((16,), N, jnp.int32)
        for _ in range(11):
            mid = low + lax.shift_right_logical(high - low, 1)
            w = lax.shift_right_logical(mid, 7)
            val = plsc.load_gather(
                bstart_v, [lax.shift_right_logical(w, 7), w & 127])
            go = r <= val
            low = jnp.where(go, low, mid)
            high = jnp.where(go, mid, high)
        rowidx_v[pl.ds(g * 16, 16)] = lax.shift_right_logical(low, 7)
        states.append((r, low, high))
    pltpu.async_copy(p_lin.at[rowidx_v], rows_v, sem).wait()

    idx_u = []
    for g in range(4):
        r, low, high = states[g]
        low0 = low
        qpos = lane + g * 16
        for _ in range(8):
            mid = low + lax.shift_right_logical(high - low, 1)
            val = plsc.load_gather(rows_v, [qpos, mid - low0])
            go = r <= val
            low = jnp.where(go, low, mid)
            high = jnp.where(go, mid, high)
        idx = high
        idx_u.append(idx)
        gx, gy = coords(idx)
        plsc.store_scatter(ptsu_b, [qpos, zero16], gx)
        plsc.store_scatter(ptsu_b, [qpos, zero16 + 1], gy)
        sdfidx_v[pl.ds(g * 16, 16)] = lax.shift_right_logical(idx, 4)

    idx_g = []
    for g in range(4):
        u = ug_v[pl.ds(g * 16, 16)]
        r = one - u
        rs = r * jnp.float32(131072.0)
        fi = lax.convert_element_type(rs, jnp.int32)
        k = fi + jnp.where(lax.convert_element_type(fi, jnp.float32) < rs, 1, 0)
        km = k - 1
        idx = lax.shift_right_logical(km, 8) * 512 + (km & 255)
        idx_g.append(idx)
        qpos = lane + g * 16
        gx, gy = coords(idx)
        plsc.store_scatter(ptsg_b, [qpos, zero16], gx)
        plsc.store_scatter(ptsg_b, [qpos, zero16 + 1], gy)
        sdfidx_v[pl.ds(64 + g * 16, 16)] = lax.shift_right_logical(idx, 4)

    for g2 in range(2):
        t = lane + g2 * 16
        sdfidx_r[pl.ds(g2 * 16, 16)] = wid * 512 + t

    pltpu.async_copy(sdf16.at[sdfidx_v], sdfrows_v, sem).wait()
    pltpu.async_copy(sdf16.at[sdfidx_r], sdfrows_r, sem).wait()

    for g in range(4):
        qpos = lane + g * 16
        val = plsc.load_gather(sdfrows_v, [qpos, idx_u[g] & 15])
        plsc.store_scatter(sdfu_b, [qpos, zero16], val)
        valg = plsc.load_gather(sdfrows_v, [qpos + 64, idx_g[g] & 15])
        plsc.store_scatter(sdfg_b, [qpos, zero16], valg)
    for g2 in range(2):
        t = lane + g2 * 16
        valr = plsc.load_gather(sdfrows_r, [t, zero16])
        plsc.store_scatter(sdfr_b, [t, zero16], valr)

    pltpu.sync_copy(ptsu_b, ptsu.at[pl.ds(base64, 64)])
    pltpu.sync_copy(sdfu_b, sdfu.at[pl.ds(base64, 64)])
    pltpu.sync_copy(ptsg_b, ptsg.at[pl.ds(base64, 64)])
    pltpu.sync_copy(sdfg_b, sdfg.at[pl.ds(base64, 64)])
    pltpu.sync_copy(sdfr_b, sdfr.at[pl.ds(wid * 32, 32)])


def kernel(sdf_map):
    sdf_flat = sdf_map.reshape(-1)

    (p_lin, bstart, plast2d, mask2d, pix2d, ptsr2d,
     sdfc2d) = _blocked_cumsum(sdf_flat.reshape(NB, BL))
    plast = plast2d.reshape(16)

    u_u = jax.random.uniform(jax.random.fold_in(jax.random.key(1), 0),
                             (2048,), dtype=jnp.float32)
    u_g = jax.random.uniform(jax.random.fold_in(jax.random.key(1), 1),
                             (2048,), dtype=jnp.float32)

    sdf16 = sdf_flat.reshape(N // 16, 16)
    pts_u, sdf_u, pts_g, sdf_g, sdf_r = _sc_sample_kernel()(
        p_lin, bstart, plast, u_u, u_g, sdf16)

    mask_for_point = mask2d.reshape(-1)
    pixels = pix2d.reshape(-1, 2)
    pts_r = ptsr2d.reshape(-1, 2)
    pixels_sdf_gt = sdfc2d.reshape(-1, 1)
    return (mask_for_point, pixels, pixels_sdf_gt,
            pts_u, sdf_u, pts_r, sdf_r, pts_g, sdf_g)


# speedup vs baseline: 5.1266x; 5.1266x over previous
import functools

import jax
import jax.numpy as jnp
from jax import lax
from jax.experimental import pallas as pl
from jax.experimental.pallas import tpu as pltpu
from jax.experimental.pallas import tpu_sc as plsc

H = W = 512
N = H * W
BL = 128
NB = N // BL
NW = 32


def _scan_body(x_ref, plin_ref, bstart_ref, plast_ref, xt_ref, p_ref, l2_ref):
    for t in range(16):
        xt_ref[:, t, :] = jnp.swapaxes(x_ref[pl.ds(t * BL, BL), :], 0, 1)
    x = xt_ref[...]
    l_idx = lax.broadcasted_iota(jnp.int32, (BL, 16, BL), 2)
    maskb = (l_idx % 4) < 2
    valid = jnp.where((jnp.abs(x) < jnp.float32(0.1)) & maskb,
                      jnp.float32(1.0), jnp.float32(0.0))
    s_total = jnp.sum(valid)
    probs = valid / jnp.maximum(s_total, jnp.float32(1e-12))

    s = jnp.zeros((16, BL), jnp.float32)
    for j in range(BL):
        s = s + probs[j]
        p_ref[j] = s

    s2d = p_ref[127]
    s2 = jnp.zeros((16, 1), jnp.float32)
    for l in range(BL):
        s2 = s2 + s2d[:, l:l + 1]
        l2_ref[:, l:l + 1] = s2

    rs3 = l2_ref[:, 127:128]
    row_i = lax.broadcasted_iota(jnp.int32, (16, 1), 0)
    carry = jnp.float32(0.0)
    off3 = jnp.zeros((16, 1), jnp.float32)
    for k in range(16):
        carry = carry + jnp.sum(jnp.where(row_i == k, rs3, jnp.float32(0.0)))
        off3 = jnp.where(row_i == k, carry, off3)

    off2 = jnp.concatenate([jnp.zeros((1, 1), jnp.float32), off3[:15, :]], axis=0)
    off_incl = l2_ref[...] + off2
    prev_last = off_incl[:, 127:128]
    prev_shift = jnp.concatenate(
        [jnp.zeros((1, 1), jnp.float32), prev_last[:15, :]], axis=0)
    off1 = jnp.concatenate([prev_shift, off_incl[:, :127]], axis=1)
    pfin = p_ref[...] + off1[None, :, :]

    for t in range(16):
        plin_ref[pl.ds(t * BL, BL), :] = jnp.swapaxes(pfin[:, t, :], 0, 1)
    bstart_ref[...] = pfin[0]
    plast_ref[...] = jnp.broadcast_to(pfin[127:128, 15, 127:128], (1, 16))


def _blocked_cumsum(sdf2d):
    return pl.pallas_call(
        _scan_body,
        out_shape=[
            jax.ShapeDtypeStruct((NB, BL), jnp.float32),
            jax.ShapeDtypeStruct((16, BL), jnp.float32),
            jax.ShapeDtypeStruct((1, 16), jnp.float32),
        ],
        scratch_shapes=[pltpu.VMEM((BL, 16, BL), jnp.float32),
                        pltpu.VMEM((BL, 16, BL), jnp.float32),
                        pltpu.VMEM((16, BL), jnp.float32)],
    )(sdf2d)


@functools.lru_cache(maxsize=1)
def _sc_sample_kernel():
  mesh = plsc.VectorSubcoreMesh(core_axis_name="c", subcore_axis_name="s")
  return functools.partial(
    pl.kernel,
    mesh=mesh,
    compiler_params=pltpu.CompilerParams(
        needs_layout_passes=False, use_tc_tiling_on_sc=False),
    out_type=[
        jax.ShapeDtypeStruct((2048, 2), jnp.float32),
        jax.ShapeDtypeStruct((2048, 1), jnp.float32),
        jax.ShapeDtypeStruct((2048, 2), jnp.float32),
        jax.ShapeDtypeStruct((2048, 1), jnp.float32),
        jax.ShapeDtypeStruct((1024, 1), jnp.float32),
    ],
    scratch_types=[
        pltpu.VMEM((16, 128), jnp.float32),
        pltpu.VMEM((16,), jnp.float32),
        pltpu.VMEM((64,), jnp.float32),
        pltpu.VMEM((64,), jnp.float32),
        pltpu.VMEM((64,), jnp.int32),
        pltpu.VMEM((64, BL), jnp.float32),
        pltpu.VMEM((64,), jnp.int32),
        pltpu.VMEM((64, 16), jnp.float32),
        pltpu.VMEM((64,), jnp.int32),
        pltpu.VMEM((64, 16), jnp.float32),
        pltpu.VMEM((32,), jnp.int32),
        pltpu.VMEM((32, 16), jnp.float32),
        pltpu.VMEM((64, 2), jnp.float32),
        pltpu.VMEM((64, 1), jnp.float32),
        pltpu.VMEM((64, 2), jnp.float32),
        pltpu.VMEM((64, 1), jnp.float32),
        pltpu.VMEM((32, 1), jnp.float32),
    ] + [pltpu.SemaphoreType.DMA] * 13,
  )(_sc_sample_body)


def _sc_sample_body(p_lin, bstart, plast, uu, ug, sdf16,
               ptsu, sdfu, ptsg, sdfg, sdfr,
               bstart_v, plast_v, uu_v, ug_v, rowidx_v, rows_v,
               sdfidx_u, sdfrows_u, sdfidx_g, sdfrows_g, sdfidx_r, sdfrows_r,
               ptsu_b, sdfu_b, ptsg_b, sdfg_b, sdfr_b,
               s_bs, s_pl, s_uu, s_ug, s_rows, s_su, s_sg, s_sr,
               s_o0, s_o1, s_o2, s_o3, s_o4):
    wid = lax.axis_index("s") * 2 + lax.axis_index("c")
    base64 = wid * 64

    c_bs = pltpu.async_copy(bstart, bstart_v, s_bs)
    c_pl = pltpu.async_copy(plast, plast_v, s_pl)
    c_uu = pltpu.async_copy(uu.at[pl.ds(base64, 64)], uu_v, s_uu)
    c_ug = pltpu.async_copy(ug.at[pl.ds(base64, 64)], ug_v, s_ug)

    lane = lax.iota(jnp.int32, 16)
    zero16 = jnp.zeros((16,), jnp.int32)
    one = jnp.float32(1.0)
    half = jnp.float32(0.5)
    inv256 = jnp.float32(1.0 / 256.0)

    def coords(idx):
        colf = lax.convert_element_type(idx & 511, jnp.float32)
        rowf = lax.convert_element_type(lax.shift_right_logical(idx, 9),
                                        jnp.float32)
        gx = (colf + half) * inv256 - one
        gy = (rowf + half) * inv256 - one
        return gx, gy

    for g2 in range(2):
        t = lane + g2 * 16
        sdfidx_r[pl.ds(g2 * 16, 16)] = wid * 512 + t
    c_sr = pltpu.async_copy(sdf16.at[sdfidx_r], sdfrows_r, s_sr)

    c_ug.wait()
    idx_g = []
    for g in range(4):
        u = ug_v[pl.ds(g * 16, 16)]
        r = one - u
        rs = r * jnp.float32(131072.0)
        fi = lax.convert_element_type(rs, jnp.int32)
        k = fi + jnp.where(lax.convert_element_type(fi, jnp.float32) < rs, 1, 0)
        km = k - 1
        idx = lax.shift_right_logical(km, 8) * 512 + (km & 255)
        idx_g.append(idx)
        qpos = lane + g * 16
        gx, gy = coords(idx)
        plsc.store_scatter(ptsg_b, [qpos, zero16], gx)
        plsc.store_scatter(ptsg_b, [qpos, zero16 + 1], gy)
        sdfidx_g[pl.ds(g * 16, 16)] = lax.shift_right_logical(idx, 4)
    c_sg = pltpu.async_copy(sdf16.at[sdfidx_g], sdfrows_g, s_sg)
    c_o2 = pltpu.async_copy(ptsg_b, ptsg.at[pl.ds(base64, 64)], s_o2)

    c_bs.wait()
    c_pl.wait()
    c_uu.wait()
    plastv = plast_v[...]
    states = []
    for g in range(4):
        u = uu_v[pl.ds(g * 16, 16)]
        r = plastv * (one - u)
        low = jnp.zeros((16,), jnp.int32)
        high = jnp.full((16,), N, jnp.int32)
        for _ in range(11):
            mid = low + lax.shift_right_logical(high - low, 1)
            w = lax.shift_right_logical(mid, 7)
            val = plsc.load_gather(
                bstart_v, [lax.shift_right_logical(w, 7), w & 127])
            go = r <= val
            low = jnp.where(go, low, mid)
            high = jnp.where(go, mid, high)
        rowidx_v[pl.ds(g * 16, 16)] = lax.shift_right_logical(low, 7)
        states.append((r, low, high))
    pltpu.async_copy(p_lin.at[rowidx_v], rows_v, s_rows).wait()

    idx_u = []
    for g in range(4):
        r, low, high = states[g]
        low0 = low
        qpos = lane + g * 16
        for _ in range(8):
            mid = low + lax.shift_right_logical(high - low, 1)
            val = plsc.load_gather(rows_v, [qpos, mid - low0])
            go = r <= val
            low = jnp.where(go, low, mid)
            high = jnp.where(go, mid, high)
        idx = high
        idx_u.append(idx)
        gx, gy = coords(idx)
        plsc.store_scatter(ptsu_b, [qpos, zero16], gx)
        plsc.store_scatter(ptsu_b, [qpos, zero16 + 1], gy)
        sdfidx_u[pl.ds(g * 16, 16)] = lax.shift_right_logical(idx, 4)
    c_su = pltpu.async_copy(sdf16.at[sdfidx_u], sdfrows_u, s_su)
    c_o0 = pltpu.async_copy(ptsu_b, ptsu.at[pl.ds(base64, 64)], s_o0)

    c_sr.wait()
    for g2 in range(2):
        t = lane + g2 * 16
        valr = plsc.load_gather(sdfrows_r, [t, zero16])
        plsc.store_scatter(sdfr_b, [t, zero16], valr)
    c_o4 = pltpu.async_copy(sdfr_b, sdfr.at[pl.ds(wid * 32, 32)], s_o4)

    c_sg.wait()
    for g in range(4):
        qpos = lane + g * 16
        valg = plsc.load_gather(sdfrows_g, [qpos, idx_g[g] & 15])
        plsc.store_scatter(sdfg_b, [qpos, zero16], valg)
    c_o3 = pltpu.async_copy(sdfg_b, sdfg.at[pl.ds(base64, 64)], s_o3)

    c_su.wait()
    for g in range(4):
        qpos = lane + g * 16
        val = plsc.load_gather(sdfrows_u, [qpos, idx_u[g] & 15])
        plsc.store_scatter(sdfu_b, [qpos, zero16], val)
    c_o1 = pltpu.async_copy(sdfu_b, sdfu.at[pl.ds(base64, 64)], s_o1)

    c_o2.wait()
    c_o0.wait()
    c_o4.wait()
    c_o3.wait()
    c_o1.wait()


def kernel(sdf_map):
    sdf_flat = sdf_map.reshape(-1)

    p_lin, bstart, plast2d = _blocked_cumsum(sdf_flat.reshape(NB, BL))
    plast = plast2d.reshape(16)

    u_u = jax.random.uniform(jax.random.fold_in(jax.random.key(1), 0),
                             (2048,), dtype=jnp.float32)
    u_g = jax.random.uniform(jax.random.fold_in(jax.random.key(1), 1),
                             (2048,), dtype=jnp.float32)

    sdf16 = sdf_flat.reshape(N // 16, 16)
    pts_u, sdf_u, pts_g, sdf_g, sdf_r = _sc_sample_kernel()(
        p_lin, bstart, plast, u_u, u_g, sdf16)

    ys = (jnp.arange(H, dtype=jnp.float32) + 0.5) / H * 2.0 - 1.0
    xs = (jnp.arange(W, dtype=jnp.float32) + 0.5) / W * 2.0 - 1.0
    gy, gx = jnp.meshgrid(ys, xs, indexing="ij")
    pixels_grid = jnp.stack([gx, gy], axis=-1)
    pixels = pixels_grid.reshape(-1, 2)
    mask_for_point = jnp.concatenate(
        (jnp.ones((H, W // 2), jnp.float32),
         jnp.zeros((H, W - W // 2), jnp.float32)), axis=-1).reshape(-1)
    pts_r = pixels_grid[::16, ::16].reshape(-1, 2)[:1024]

    pixels_sdf_gt = sdf_map.reshape(-1, 1)
    return (mask_for_point, pixels, pixels_sdf_gt,
            pts_u, sdf_u, pts_r, sdf_r, pts_g, sdf_g)


# speedup vs baseline: 5.4764x; 1.0682x over previous
import functools

import jax
import jax.numpy as jnp
import numpy as np
from jax import lax
from jax.experimental import pallas as pl
from jax.experimental.pallas import tpu as pltpu
from jax.experimental.pallas import tpu_sc as plsc

H = W = 512
N = H * W
BL = 128
NB = N // BL
NW = 32


def _host_constants():
    ar = np.arange(H, dtype=np.float32)
    ys = (ar + np.float32(0.5)) / np.float32(H) * np.float32(2) - np.float32(1)
    gx = np.broadcast_to(ys[None, :], (H, W))
    gy = np.broadcast_to(ys[:, None], (H, W))
    pixels_grid = np.stack([gx, gy], axis=-1)
    pixels = np.ascontiguousarray(pixels_grid.reshape(-1, 2))
    mask = np.zeros((H, W), np.float32)
    mask[:, :W // 2] = np.float32(1.0)
    mask = mask.reshape(-1)
    pts_r = np.ascontiguousarray(
        pixels_grid[::16, ::16].reshape(-1, 2)[:1024])
    cpu = jax.devices("cpu")[0]
    with jax.default_device(cpu):
        uu = np.asarray(jax.random.uniform(
            jax.random.fold_in(jax.random.key(1), 0), (2048,),
            dtype=jnp.float32))
        ug = np.asarray(jax.random.uniform(
            jax.random.fold_in(jax.random.key(1), 1), (2048,),
            dtype=jnp.float32))
    return mask, pixels, pts_r, uu, ug


_MASK_C, _PIXELS_C, _PTSR_C, _UU_C, _UG_C = _host_constants()


def _scan_body(x_ref, plin_ref, bstart_ref, plast_ref, xt_ref, p_ref, l2_ref):
    for t in range(16):
        xt_ref[:, t, :] = jnp.swapaxes(x_ref[pl.ds(t * BL, BL), :], 0, 1)
    x = xt_ref[...]
    l_idx = lax.broadcasted_iota(jnp.int32, (BL, 16, BL), 2)
    maskb = (l_idx % 4) < 2
    valid = jnp.where((jnp.abs(x) < jnp.float32(0.1)) & maskb,
                      jnp.float32(1.0), jnp.float32(0.0))
    s_total = jnp.sum(valid)
    probs = valid / jnp.maximum(s_total, jnp.float32(1e-12))

    s = jnp.zeros((16, BL), jnp.float32)
    for j in range(BL):
        s = s + probs[j]
        p_ref[j] = s

    s2d = p_ref[127]
    s2 = jnp.zeros((16, 1), jnp.float32)
    for l in range(BL):
        s2 = s2 + s2d[:, l:l + 1]
        l2_ref[:, l:l + 1] = s2

    rs3 = l2_ref[:, 127:128]
    row_i = lax.broadcasted_iota(jnp.int32, (16, 1), 0)
    carry = jnp.float32(0.0)
    off3 = jnp.zeros((16, 1), jnp.float32)
    for k in range(16):
        carry = carry + jnp.sum(jnp.where(row_i == k, rs3, jnp.float32(0.0)))
        off3 = jnp.where(row_i == k, carry, off3)

    off2 = jnp.concatenate([jnp.zeros((1, 1), jnp.float32), off3[:15, :]], axis=0)
    off_incl = l2_ref[...] + off2
    prev_last = off_incl[:, 127:128]
    prev_shift = jnp.concatenate(
        [jnp.zeros((1, 1), jnp.float32), prev_last[:15, :]], axis=0)
    off1 = jnp.concatenate([prev_shift, off_incl[:, :127]], axis=1)
    pfin = p_ref[...] + off1[None, :, :]

    for t in range(16):
        plin_ref[pl.ds(t * BL, BL), :] = jnp.swapaxes(pfin[:, t, :], 0, 1)
    bstart_ref[...] = pfin[0]
    plast_ref[...] = jnp.broadcast_to(pfin[127:128, 15, 127:128], (1, 16))


def _blocked_cumsum(sdf2d):
    return pl.pallas_call(
        _scan_body,
        out_shape=[
            jax.ShapeDtypeStruct((NB, BL), jnp.float32),
            jax.ShapeDtypeStruct((16, BL), jnp.float32),
            jax.ShapeDtypeStruct((1, 16), jnp.float32),
        ],
        scratch_shapes=[pltpu.VMEM((BL, 16, BL), jnp.float32),
                        pltpu.VMEM((BL, 16, BL), jnp.float32),
                        pltpu.VMEM((16, BL), jnp.float32)],
    )(sdf2d)


@functools.lru_cache(maxsize=1)
def _sc_sample_kernel():
  mesh = plsc.VectorSubcoreMesh(core_axis_name="c", subcore_axis_name="s")
  return functools.partial(
    pl.kernel,
    mesh=mesh,
    compiler_params=pltpu.CompilerParams(
        needs_layout_passes=False, use_tc_tiling_on_sc=False),
    out_type=[
        jax.ShapeDtypeStruct((2048, 2), jnp.float32),
        jax.ShapeDtypeStruct((2048, 1), jnp.float32),
        jax.ShapeDtypeStruct((2048, 2), jnp.float32),
        jax.ShapeDtypeStruct((2048, 1), jnp.float32),
        jax.ShapeDtypeStruct((1024, 1), jnp.float32),
    ],
    scratch_types=[
        pltpu.VMEM((16, 128), jnp.float32),
        pltpu.VMEM((16,), jnp.float32),
        pltpu.VMEM((64,), jnp.float32),
        pltpu.VMEM((64,), jnp.float32),
        pltpu.VMEM((64,), jnp.int32),
        pltpu.VMEM((64, BL), jnp.float32),
        pltpu.VMEM((64,), jnp.int32),
        pltpu.VMEM((64, 16), jnp.float32),
        pltpu.VMEM((64,), jnp.int32),
        pltpu.VMEM((64, 16), jnp.float32),
        pltpu.VMEM((32,), jnp.int32),
        pltpu.VMEM((32, 16), jnp.float32),
        pltpu.VMEM((64, 2), jnp.float32),
        pltpu.VMEM((64, 1), jnp.float32),
        pltpu.VMEM((64, 2), jnp.float32),
        pltpu.VMEM((64, 1), jnp.float32),
        pltpu.VMEM((32, 1), jnp.float32),
    ] + [pltpu.SemaphoreType.DMA] * 13,
  )(_sc_sample_body)


def _sc_sample_body(p_lin, bstart, plast, uu, ug, sdf16,
               ptsu, sdfu, ptsg, sdfg, sdfr,
               bstart_v, plast_v, uu_v, ug_v, rowidx_v, rows_v,
               sdfidx_u, sdfrows_u, sdfidx_g, sdfrows_g, sdfidx_r, sdfrows_r,
               ptsu_b, sdfu_b, ptsg_b, sdfg_b, sdfr_b,
               s_bs, s_pl, s_uu, s_ug, s_rows, s_su, s_sg, s_sr,
               s_o0, s_o1, s_o2, s_o3, s_o4):
    wid = lax.axis_index("s") * 2 + lax.axis_index("c")
    base64 = wid * 64

    c_bs = pltpu.async_copy(bstart, bstart_v, s_bs)
    c_pl = pltpu.async_copy(plast, plast_v, s_pl)
    c_uu = pltpu.async_copy(uu.at[pl.ds(base64, 64)], uu_v, s_uu)
    c_ug = pltpu.async_copy(ug.at[pl.ds(base64, 64)], ug_v, s_ug)

    lane = lax.iota(jnp.int32, 16)
    zero16 = jnp.zeros((16,), jnp.int32)
    one = jnp.float32(1.0)
    half = jnp.float32(0.5)
    inv256 = jnp.float32(1.0 / 256.0)

    def coords(idx):
        colf = lax.convert_element_type(idx & 511, jnp.float32)
        rowf = lax.convert_element_type(lax.shift_right_logical(idx, 9),
                                        jnp.float32)
        gx = (colf + half) * inv256 - one
        gy = (rowf + half) * inv256 - one
        return gx, gy

    for g2 in range(2):
        t = lane + g2 * 16
        sdfidx_r[pl.ds(g2 * 16, 16)] = wid * 512 + t
    c_sr = pltpu.async_copy(sdf16.at[sdfidx_r], sdfrows_r, s_sr)

    c_ug.wait()
    idx_g = []
    for g in range(4):
        u = ug_v[pl.ds(g * 16, 16)]
        r = one - u
        rs = r * jnp.float32(131072.0)
        fi = lax.convert_element_type(rs, jnp.int32)
        k = fi + jnp.where(lax.convert_element_type(fi, jnp.float32) < rs, 1, 0)
        km = k - 1
        idx = lax.shift_right_logical(km, 8) * 512 + (km & 255)
        idx_g.append(idx)
        qpos = lane + g * 16
        gx, gy = coords(idx)
        plsc.store_scatter(ptsg_b, [qpos, zero16], gx)
        plsc.store_scatter(ptsg_b, [qpos, zero16 + 1], gy)
        sdfidx_g[pl.ds(g * 16, 16)] = lax.shift_right_logical(idx, 4)
    c_sg = pltpu.async_copy(sdf16.at[sdfidx_g], sdfrows_g, s_sg)
    c_o2 = pltpu.async_copy(ptsg_b, ptsg.at[pl.ds(base64, 64)], s_o2)

    c_bs.wait()
    c_pl.wait()
    c_uu.wait()
    plastv = plast_v[...]
    states = []
    for g in range(4):
        u = uu_v[pl.ds(g * 16, 16)]
        r = plastv * (one - u)
        low = jnp.zeros((16,), jnp.int32)
        high = jnp.full((16,), N, jnp.int32)
        for _ in range(11):
            mid = low + lax.shift_right_logical(high - low, 1)
            w = lax.shift_right_logical(mid, 7)
            val = plsc.load_gather(
                bstart_v, [lax.shift_right_logical(w, 7), w & 127])
            go = r <= val
            low = jnp.where(go, low, mid)
            high = jnp.where(go, mid, high)
        rowidx_v[pl.ds(g * 16, 16)] = lax.shift_right_logical(low, 7)
        states.append((r, low, high))
    pltpu.async_copy(p_lin.at[rowidx_v], rows_v, s_rows).wait()

    idx_u = []
    for g in range(4):
        r, low, high = states[g]
        low0 = low
        qpos = lane + g * 16
        for _ in range(8):
            mid = low + lax.shift_right_logical(high - low, 1)
            val = plsc.load_gather(rows_v, [qpos, mid - low0])
            go = r <= val
            low = jnp.where(go, low, mid)
            high = jnp.where(go, mid, high)
        idx = high
        idx_u.append(idx)
        gx, gy = coords(idx)
        plsc.store_scatter(ptsu_b, [qpos, zero16], gx)
        plsc.store_scatter(ptsu_b, [qpos, zero16 + 1], gy)
        sdfidx_u[pl.ds(g * 16, 16)] = lax.shift_right_logical(idx, 4)
    c_su = pltpu.async_copy(sdf16.at[sdfidx_u], sdfrows_u, s_su)
    c_o0 = pltpu.async_copy(ptsu_b, ptsu.at[pl.ds(base64, 64)], s_o0)

    c_sr.wait()
    for g2 in range(2):
        t = lane + g2 * 16
        valr = plsc.load_gather(sdfrows_r, [t, zero16])
        plsc.store_scatter(sdfr_b, [t, zero16], valr)
    c_o4 = pltpu.async_copy(sdfr_b, sdfr.at[pl.ds(wid * 32, 32)], s_o4)

    c_sg.wait()
    for g in range(4):
        qpos = lane + g * 16
        valg = plsc.load_gather(sdfrows_g, [qpos, idx_g[g] & 15])
        plsc.store_scatter(sdfg_b, [qpos, zero16], valg)
    c_o3 = pltpu.async_copy(sdfg_b, sdfg.at[pl.ds(base64, 64)], s_o3)

    c_su.wait()
    for g in range(4):
        qpos = lane + g * 16
        val = plsc.load_gather(sdfrows_u, [qpos, idx_u[g] & 15])
        plsc.store_scatter(sdfu_b, [qpos, zero16], val)
    c_o1 = pltpu.async_copy(sdfu_b, sdfu.at[pl.ds(base64, 64)], s_o1)

    c_o2.wait()
    c_o0.wait()
    c_o4.wait()
    c_o3.wait()
    c_o1.wait()


def kernel(sdf_map):
    sdf_flat = sdf_map.reshape(-1)

    p_lin, bstart, plast2d = _blocked_cumsum(sdf_flat.reshape(NB, BL))
    plast = plast2d.reshape(16)

    u_u = jnp.asarray(_UU_C)
    u_g = jnp.asarray(_UG_C)

    sdf16 = sdf_flat.reshape(N // 16, 16)
    pts_u, sdf_u, pts_g, sdf_g, sdf_r = _sc_sample_kernel()(
        p_lin, bstart, plast, u_u, u_g, sdf16)

    mask_for_point = jnp.asarray(_MASK_C)
    pixels = jnp.asarray(_PIXELS_C)
    pts_r = jnp.asarray(_PTSR_C)

    pixels_sdf_gt = sdf_map.reshape(-1, 1)
    return (mask_for_point, pixels, pixels_sdf_gt,
            pts_u, sdf_u, pts_r, sdf_r, pts_g, sdf_g)


# speedup vs baseline: 5.9732x; 1.0907x over previous
import functools

import jax
import jax.numpy as jnp
import numpy as np
from jax import lax
from jax.experimental import pallas as pl
from jax.experimental.pallas import tpu as pltpu
from jax.experimental.pallas import tpu_sc as plsc

H = W = 512
N = H * W
BL = 128
NB = N // BL
NW = 32


def _host_constants():
    ar = np.arange(H, dtype=np.float32)
    ys = (ar + np.float32(0.5)) / np.float32(H) * np.float32(2) - np.float32(1)
    gx = np.broadcast_to(ys[None, :], (H, W))
    gy = np.broadcast_to(ys[:, None], (H, W))
    pixels_grid = np.stack([gx, gy], axis=-1)
    pixels = np.ascontiguousarray(pixels_grid.reshape(-1, 2))
    mask = np.zeros((H, W), np.float32)
    mask[:, :W // 2] = np.float32(1.0)
    mask = mask.reshape(-1)
    pts_r = np.ascontiguousarray(
        pixels_grid[::16, ::16].reshape(-1, 2)[:1024])
    cpu = jax.devices("cpu")[0]
    with jax.default_device(cpu):
        uu = np.asarray(jax.random.uniform(
            jax.random.fold_in(jax.random.key(1), 0), (2048,),
            dtype=jnp.float32))
        ug = np.asarray(jax.random.uniform(
            jax.random.fold_in(jax.random.key(1), 1), (2048,),
            dtype=jnp.float32))
    return mask, pixels, pts_r, uu, ug


_MASK_C, _PIXELS_C, _PTSR_C, _UU_C, _UG_C = _host_constants()


def _scan_body(x_ref, plin_ref, bstart_ref, plast_ref, xt_ref, p_ref, l2_ref):
    for t in range(16):
        xt_ref[:, t, :] = jnp.swapaxes(x_ref[pl.ds(t * BL, BL), :], 0, 1)
    x = xt_ref[...]
    l_idx = lax.broadcasted_iota(jnp.int32, (BL, 16, BL), 2)
    maskb = (l_idx % 4) < 2
    valid = jnp.where((jnp.abs(x) < jnp.float32(0.1)) & maskb,
                      jnp.float32(1.0), jnp.float32(0.0))
    s_total = jnp.sum(valid)
    probs = valid / jnp.maximum(s_total, jnp.float32(1e-12))

    s = jnp.zeros((16, BL), jnp.float32)
    for j in range(BL):
        s = s + probs[j]
        p_ref[j] = s

    s2d = p_ref[127]
    s2 = jnp.zeros((16, 1), jnp.float32)
    for l in range(BL):
        s2 = s2 + s2d[:, l:l + 1]
        l2_ref[:, l:l + 1] = s2

    rs3 = l2_ref[:, 127:128]
    row_i = lax.broadcasted_iota(jnp.int32, (16, 1), 0)
    carry = jnp.float32(0.0)
    off3 = jnp.zeros((16, 1), jnp.float32)
    for k in range(16):
        carry = carry + jnp.sum(jnp.where(row_i == k, rs3, jnp.float32(0.0)))
        off3 = jnp.where(row_i == k, carry, off3)

    off2 = jnp.concatenate([jnp.zeros((1, 1), jnp.float32), off3[:15, :]], axis=0)
    off_incl = l2_ref[...] + off2
    prev_last = off_incl[:, 127:128]
    prev_shift = jnp.concatenate(
        [jnp.zeros((1, 1), jnp.float32), prev_last[:15, :]], axis=0)
    off1 = jnp.concatenate([prev_shift, off_incl[:, :127]], axis=1)
    pfin = p_ref[...] + off1[None, :, :]

    for t in range(16):
        plin_ref[pl.ds(t * BL, BL), :] = jnp.swapaxes(pfin[:, t, :], 0, 1)
    bstart_ref[...] = pfin[0]
    plast_ref[...] = jnp.broadcast_to(pfin[127:128, 15, 127:128], (1, 16))


def _blocked_cumsum(sdf2d):
    return pl.pallas_call(
        _scan_body,
        out_shape=[
            jax.ShapeDtypeStruct((NB, BL), jnp.float32),
            jax.ShapeDtypeStruct((16, BL), jnp.float32),
            jax.ShapeDtypeStruct((1, 16), jnp.float32),
        ],
        scratch_shapes=[pltpu.VMEM((BL, 16, BL), jnp.float32),
                        pltpu.VMEM((BL, 16, BL), jnp.float32),
                        pltpu.VMEM((16, BL), jnp.float32)],
    )(sdf2d)


_SC_MESH = plsc.VectorSubcoreMesh(core_axis_name="c", subcore_axis_name="s")
_SC_PARAMS = pltpu.CompilerParams(
    needs_layout_passes=False, use_tc_tiling_on_sc=False)


@functools.lru_cache(maxsize=1)
def _sc_global_kernel():
  return functools.partial(
    pl.kernel,
    mesh=_SC_MESH,
    compiler_params=_SC_PARAMS,
    out_type=[
        jax.ShapeDtypeStruct((2048, 2), jnp.float32),
        jax.ShapeDtypeStruct((2048, 1), jnp.float32),
        jax.ShapeDtypeStruct((1024, 1), jnp.float32),
    ],
    scratch_types=[
        pltpu.VMEM((64,), jnp.float32),
        pltpu.VMEM((64,), jnp.int32),
        pltpu.VMEM((64, 16), jnp.float32),
        pltpu.VMEM((32,), jnp.int32),
        pltpu.VMEM((32, 16), jnp.float32),
        pltpu.VMEM((64, 2), jnp.float32),
        pltpu.VMEM((64, 1), jnp.float32),
        pltpu.VMEM((32, 1), jnp.float32),
    ] + [pltpu.SemaphoreType.DMA] * 6,
  )(_sc_global_body)


def _sc_coords(idx):
    one = jnp.float32(1.0)
    half = jnp.float32(0.5)
    inv256 = jnp.float32(1.0 / 256.0)
    colf = lax.convert_element_type(idx & 511, jnp.float32)
    rowf = lax.convert_element_type(lax.shift_right_logical(idx, 9),
                                    jnp.float32)
    gx = (colf + half) * inv256 - one
    gy = (rowf + half) * inv256 - one
    return gx, gy


def _sc_global_body(ug, sdf16, ptsg, sdfg, sdfr,
                    ug_v, sdfidx_g, sdfrows_g, sdfidx_r, sdfrows_r,
                    ptsg_b, sdfg_b, sdfr_b,
                    s_ug, s_sg, s_sr, s_o2, s_o3, s_o4):
    wid = lax.axis_index("s") * 2 + lax.axis_index("c")
    base64 = wid * 64
    c_ug = pltpu.async_copy(ug.at[pl.ds(base64, 64)], ug_v, s_ug)

    lane = lax.iota(jnp.int32, 16)
    zero16 = jnp.zeros((16,), jnp.int32)
    one = jnp.float32(1.0)

    for g2 in range(2):
        t = lane + g2 * 16
        sdfidx_r[pl.ds(g2 * 16, 16)] = wid * 512 + t
    c_sr = pltpu.async_copy(sdf16.at[sdfidx_r], sdfrows_r, s_sr)

    c_ug.wait()
    idx_g = []
    for g in range(4):
        u = ug_v[pl.ds(g * 16, 16)]
        r = one - u
        rs = r * jnp.float32(131072.0)
        fi = lax.convert_element_type(rs, jnp.int32)
        k = fi + jnp.where(lax.convert_element_type(fi, jnp.float32) < rs, 1, 0)
        km = k - 1
        idx = lax.shift_right_logical(km, 8) * 512 + (km & 255)
        idx_g.append(idx)
        qpos = lane + g * 16
        gx, gy = _sc_coords(idx)
        plsc.store_scatter(ptsg_b, [qpos, zero16], gx)
        plsc.store_scatter(ptsg_b, [qpos, zero16 + 1], gy)
        sdfidx_g[pl.ds(g * 16, 16)] = lax.shift_right_logical(idx, 4)
    c_sg = pltpu.async_copy(sdf16.at[sdfidx_g], sdfrows_g, s_sg)
    c_o2 = pltpu.async_copy(ptsg_b, ptsg.at[pl.ds(base64, 64)], s_o2)

    c_sr.wait()
    for g2 in range(2):
        t = lane + g2 * 16
        valr = plsc.load_gather(sdfrows_r, [t, zero16])
        plsc.store_scatter(sdfr_b, [t, zero16], valr)
    c_o4 = pltpu.async_copy(sdfr_b, sdfr.at[pl.ds(wid * 32, 32)], s_o4)

    c_sg.wait()
    for g in range(4):
        qpos = lane + g * 16
        valg = plsc.load_gather(sdfrows_g, [qpos, idx_g[g] & 15])
        plsc.store_scatter(sdfg_b, [qpos, zero16], valg)
    c_o3 = pltpu.async_copy(sdfg_b, sdfg.at[pl.ds(base64, 64)], s_o3)

    c_o2.wait()
    c_o4.wait()
    c_o3.wait()


@functools.lru_cache(maxsize=1)
def _sc_untrunc_kernel():
  return functools.partial(
    pl.kernel,
    mesh=_SC_MESH,
    compiler_params=_SC_PARAMS,
    out_type=[
        jax.ShapeDtypeStruct((2048, 2), jnp.float32),
        jax.ShapeDtypeStruct((2048, 1), jnp.float32),
    ],
    scratch_types=[
        pltpu.VMEM((16, 128), jnp.float32),
        pltpu.VMEM((16,), jnp.float32),
        pltpu.VMEM((64,), jnp.float32),
        pltpu.VMEM((64,), jnp.int32),
        pltpu.VMEM((64, BL), jnp.float32),
        pltpu.VMEM((64,), jnp.int32),
        pltpu.VMEM((64, 16), jnp.float32),
        pltpu.VMEM((64, 2), jnp.float32),
        pltpu.VMEM((64, 1), jnp.float32),
    ] + [pltpu.SemaphoreType.DMA] * 7,
  )(_sc_untrunc_body)


def _sc_untrunc_body(p_lin, bstart, plast, uu, sdf16,
                     ptsu, sdfu,
                     bstart_v, plast_v, uu_v, rowidx_v, rows_v,
                     sdfidx_u, sdfrows_u, ptsu_b, sdfu_b,
                     s_bs, s_pl, s_uu, s_rows, s_su, s_o0, s_o1):
    wid = lax.axis_index("s") * 2 + lax.axis_index("c")
    base64 = wid * 64

    c_bs = pltpu.async_copy(bstart, bstart_v, s_bs)
    c_pl = pltpu.async_copy(plast, plast_v, s_pl)
    c_uu = pltpu.async_copy(uu.at[pl.ds(base64, 64)], uu_v, s_uu)

    lane = lax.iota(jnp.int32, 16)
    zero16 = jnp.zeros((16,), jnp.int32)
    one = jnp.float32(1.0)

    c_bs.wait()
    c_pl.wait()
    c_uu.wait()
    plastv = plast_v[...]
    states = []
    for g in range(4):
        u = uu_v[pl.ds(g * 16, 16)]
        r = plastv * (one - u)
        low = jnp.zeros((16,), jnp.int32)
        high = jnp.full((16,), N, jnp.int32)
        for _ in range(11):
            mid = low + lax.shift_right_logical(high - low, 1)
            w = lax.shift_right_logical(mid, 7)
            val = plsc.load_gather(
                bstart_v, [lax.shift_right_logical(w, 7), w & 127])
            go = r <= val
            low = jnp.where(go, low, mid)
            high = jnp.where(go, mid, high)
        rowidx_v[pl.ds(g * 16, 16)] = lax.shift_right_logical(low, 7)
        states.append((r, low, high))
    pltpu.async_copy(p_lin.at[rowidx_v], rows_v, s_rows).wait()

    idx_u = []
    for g in range(4):
        r, low, high = states[g]
        low0 = low
        qpos = lane + g * 16
        for _ in range(8):
            mid = low + lax.shift_right_logical(high - low, 1)
            val = plsc.load_gather(rows_v, [qpos, mid - low0])
            go = r <= val
            low = jnp.where(go, low, mid)
            high = jnp.where(go, mid, high)
        idx = high
        idx_u.append(idx)
        gx, gy = _sc_coords(idx)
        plsc.store_scatter(ptsu_b, [qpos, zero16], gx)
        plsc.store_scatter(ptsu_b, [qpos, zero16 + 1], gy)
        sdfidx_u[pl.ds(g * 16, 16)] = lax.shift_right_logical(idx, 4)
    c_su = pltpu.async_copy(sdf16.at[sdfidx_u], sdfrows_u, s_su)
    c_o0 = pltpu.async_copy(ptsu_b, ptsu.at[pl.ds(base64, 64)], s_o0)

    c_su.wait()
    for g in range(4):
        qpos = lane + g * 16
        val = plsc.load_gather(sdfrows_u, [qpos, idx_u[g] & 15])
        plsc.store_scatter(sdfu_b, [qpos, zero16], val)
    c_o1 = pltpu.async_copy(sdfu_b, sdfu.at[pl.ds(base64, 64)], s_o1)

    c_o0.wait()
    c_o1.wait()


def kernel(sdf_map):
    sdf_flat = sdf_map.reshape(-1)
    sdf16 = sdf_flat.reshape(N // 16, 16)

    u_u = jnp.asarray(_UU_C)
    u_g = jnp.asarray(_UG_C)

    pts_g, sdf_g, sdf_r = _sc_global_kernel()(u_g, sdf16)

    p_lin, bstart, plast2d = _blocked_cumsum(sdf_flat.reshape(NB, BL))
    plast = plast2d.reshape(16)

    pts_u, sdf_u = _sc_untrunc_kernel()(p_lin, bstart, plast, u_u, sdf16)

    mask_for_point = jnp.asarray(_MASK_C)
    pixels = jnp.asarray(_PIXELS_C)
    pts_r = jnp.asarray(_PTSR_C)

    pixels_sdf_gt = sdf_map.reshape(-1, 1)
    return (mask_for_point, pixels, pixels_sdf_gt,
            pts_u, sdf_u, pts_r, sdf_r, pts_g, sdf_g)
